# baseline (speedup 1.0000x reference)
import functools

import jax
import jax.numpy as jnp
import numpy as np
from jax import lax
from jax.experimental import pallas as pl
from jax.experimental.pallas import tpu as pltpu
from jax.experimental.pallas import tpu_sc as plsc

N = 10000
E = 320000
D = 128

NC = 2
NS = 16
NW = NC * NS
EPW = E // NW
C = 40
C2 = 2 * C
NCHUNK = EPW // C

NPAD = 10112
RPT = NPAD // NS
NDEN = 640
DRPT = NDEN // NS

_ROW_BLK = 2000

_P = np.zeros(D, dtype=np.int32)
for _t in range(4):
    for _p in range(32):
        _P[32 * _t + _p] = 32 * _t + (2 * _p if _p < 16 else 2 * (_p - 16) + 1)
_PINV = np.argsort(_P).astype(np.int32)


def _linears_body(x_ref, wqm_ref, bqm_ref, wk_ref, bk_ref, ws_ref, bs_ref,
                  qmsg_ref, k_ref, self_ref):
    x = x_ref[...]
    qmsg_ref[...] = (jnp.dot(x, wqm_ref[...], preferred_element_type=jnp.float32)
                     + bqm_ref[...]).astype(jnp.bfloat16)
    k_ref[...] = jnp.dot(x, wk_ref[...],
                         preferred_element_type=jnp.float32) + bk_ref[...]
    self_ref[...] = jnp.dot(x, ws_ref[...],
                            preferred_element_type=jnp.float32) + bs_ref[...]


def _linears(x, wqm, bqm, wk, bk, ws, bs):
    nblk = N // _ROW_BLK
    return pl.pallas_call(
        _linears_body,
        grid=(nblk,),
        in_specs=[
            pl.BlockSpec((_ROW_BLK, D), lambda i: (i, 0)),
            pl.BlockSpec((D, 2 * D), lambda i: (0, 0)),
            pl.BlockSpec((1, 2 * D), lambda i: (0, 0)),
            pl.BlockSpec((D, D), lambda i: (0, 0)),
            pl.BlockSpec((1, D), lambda i: (0, 0)),
            pl.BlockSpec((D, D), lambda i: (0, 0)),
            pl.BlockSpec((1, D), lambda i: (0, 0)),
        ],
        out_specs=[
            pl.BlockSpec((_ROW_BLK, 2 * D), lambda i: (i, 0)),
            pl.BlockSpec((_ROW_BLK, D), lambda i: (i, 0)),
            pl.BlockSpec((_ROW_BLK, D), lambda i: (i, 0)),
        ],
        out_shape=[
            jax.ShapeDtypeStruct((N, 2 * D), jnp.bfloat16),
            jax.ShapeDtypeStruct((N, D), jnp.float32),
            jax.ShapeDtypeStruct((N, D), jnp.float32),
        ],
    )(x, wqm, bqm, wk, bk, ws, bs)


_GDN = lax.GatherDimensionNumbers(offset_dims=(), collapsed_slice_dims=(0,),
                                  start_index_map=(0,))


def _perm16(v, idx):
    return lax.gather(v, idx[:, None], _GDN, (1,), unique_indices=True,
                      mode=lax.GatherScatterMode.PROMISE_IN_BOUNDS)


def _halves(ref, e, t):
    xi = ref[e, pl.ds(16 * t, 16)]
    lo = lax.bitcast_convert_type(lax.shift_left(xi, 16), jnp.float32)
    hi = lax.bitcast_convert_type(jnp.bitwise_and(xi, jnp.int32(-65536)),
                                  jnp.float32)
    return lo, hi


_IBLK = 8
_IBUF = 2 * _IBLK


def _edge_body(qmsg_hbm, k_hbm, zeros_hbm, ei_hbm,
               acc_out_hbm, den_out_hbm,
               ibblk, oidx0, oidx1, qv0, qv1, kv0, kv1, oa0, oa1, arena,
               gq0, gq1, gk0, gk1, ss0, ss1):
    c = lax.axis_index("c")
    s = lax.axis_index("s")
    wid = s * NC + c
    row0 = wid * NCHUNK

    pltpu.sync_copy(zeros_hbm.at[pl.ds(0, RPT)],
                    arena.at[pl.ds(s * RPT, RPT)])
    pltpu.sync_copy(zeros_hbm.at[pl.ds(0, DRPT)],
                    arena.at[pl.ds(NPAD + s * DRPT, DRPT)])
    plsc.subcore_barrier()

    lane = lax.iota(jnp.int32, 16)
    perms = [lane ^ sh for sh in (1, 2, 4, 8)]
    zero16 = jnp.zeros((16,), jnp.float32)
    hi_m = lax.convert_element_type(lax.shift_right_logical(lane, 3),
                                    jnp.float32)
    mlo = 1.0 - hi_m
    mdiff = hi_m - mlo

    qvs = (qv0, qv1)
    kvs = (kv0, kv1)
    oidxs = (oidx0, oidx1)
    oas = (oa0, oa1)
    gqs = (gq0, gq1)
    gks = (gk0, gk1)
    sss = (ss0, ss1)

    def _slot(i):
        return lax.rem(i, _IBUF) * C2

    def _fetch_blk(i):
        pltpu.sync_copy(ei_hbm.at[pl.ds((row0 + i) * C2, _IBLK * C2)],
                        ibblk.at[pl.ds(_slot(i), _IBLK * C2)])

    def _gathers(i, b):
        off = _slot(i)
        return (
            pltpu.make_async_copy(qmsg_hbm.at[ibblk.at[pl.ds(off, C)]],
                                  qvs[b], gqs[b]),
            pltpu.make_async_copy(k_hbm.at[ibblk.at[pl.ds(off + C, C)]],
                                  kvs[b], gks[b]),
        )

    def _scatter_start(b):
        pltpu.async_copy(oas[b], arena.at[oidxs[b]], sss[b], add=True)

    def _scatter_wait(b):
        pltpu.make_async_copy(oas[b], arena.at[oidxs[b]], sss[b]).wait()

    def _stage(i, cur):
        nxt = 1 - cur

        @pl.when(jnp.logical_and(lax.rem(i + 1, _IBLK) == 0,
                                 i + 1 < NCHUNK))
        def _():
            _fetch_blk(i + 1)

        @pl.when(i + 1 < NCHUNK)
        def _():
            for cp in _gathers(i + 1, nxt):
                cp.start()

        @pl.when(i >= 2)
        def _():
            _scatter_wait(cur)

        for cp in _gathers(i, cur):
            cp.wait()
        off = _slot(i)
        qv = qvs[cur]
        kv = kvs[cur]
        oidx = oidxs[cur]
        o_all = oas[cur]
        for e0, u0 in ((0, 0), (16, 0), (24, 8)):
            dst16 = ibblk[pl.ds(off + C + e0, 16)]
            oidx[pl.ds(e0, 16)] = dst16
            oidx[pl.ds(C + e0, 16)] = lax.shift_right_logical(dst16, 4) + NPAD
            for u in range(u0, 16):
                e = e0 + u
                qlo, qhi = _halves(qv, e, 0)
                acc = (qlo * kv[e, pl.ds(0, 16)] +
                       qhi * kv[e, pl.ds(16, 16)])
                for t in range(1, 4):
                    qlo, qhi = _halves(qv, e, t)
                    acc = acc + (qlo * kv[e, pl.ds(32 * t, 16)] +
                                 qhi * kv[e, pl.ds(32 * t + 16, 16)])
                for p in perms:
                    acc = acc + _perm16(acc, p)
                ex = jnp.exp(acc)
                for t in range(4):
                    mlo_v, mhi_v = _halves(qv, e, 4 + t)
                    o_all[e, pl.ds(32 * t, 16)] = ex * mlo_v
                    o_all[e, pl.ds(32 * t + 16, 16)] = ex * mhi_v
                dst_s = dst16[u]
                halff = lax.convert_element_type(
                    jnp.bitwise_and(dst_s, 1), jnp.float32)
                vec = ex * (mlo + halff * mdiff)
                alig = pl.multiple_of(
                    jnp.bitwise_and(lax.shift_right_logical(dst_s, 1), 7)
                    * 16, 16)
                for j in range(8):
                    o_all[C + e, pl.ds(16 * j, 16)] = zero16
                o_all[C + e, pl.ds(alig, 16)] = vec
        _scatter_start(cur)

    _fetch_blk(0)
    for cp in _gathers(0, 0):
        cp.start()

    def pair(t, carry):
        _stage(2 * t, 0)
        _stage(2 * t + 1, 1)
        return carry

    lax.fori_loop(0, NCHUNK // 2, pair, 0)
    _scatter_wait(0)
    _scatter_wait(1)

    plsc.subcore_barrier()
    pltpu.sync_copy(arena.at[pl.ds(s * RPT, RPT)],
                    acc_out_hbm.at[c, pl.ds(s * RPT, RPT)])
    pltpu.sync_copy(arena.at[pl.ds(NPAD + s * DRPT, DRPT)],
                    den_out_hbm.at[c, pl.ds(s * DRPT, DRPT)])


_edge_kernel = functools.partial(
    pl.kernel,
    out_type=[
        jax.ShapeDtypeStruct((NC, NPAD, D), jnp.float32),
        jax.ShapeDtypeStruct((NC, NDEN, D), jnp.float32),
    ],
    mesh=plsc.VectorSubcoreMesh(core_axis_name="c", subcore_axis_name="s"),
    scratch_types=[
        pltpu.VMEM((_IBUF * C2,), jnp.int32),
        pltpu.VMEM((C2,), jnp.int32),
        pltpu.VMEM((C2,), jnp.int32),
        pltpu.VMEM((C, D), jnp.int32),
        pltpu.VMEM((C, D), jnp.int32),
        pltpu.VMEM((C, D), jnp.float32),
        pltpu.VMEM((C, D), jnp.float32),
        pltpu.VMEM((C2, D), jnp.float32),
        pltpu.VMEM((C2, D), jnp.float32),
        pltpu.VMEM_SHARED((NPAD + NDEN, D), jnp.float32),
        pltpu.SemaphoreType.DMA,
        pltpu.SemaphoreType.DMA,
        pltpu.SemaphoreType.DMA,
        pltpu.SemaphoreType.DMA,
        pltpu.SemaphoreType.DMA,
        pltpu.SemaphoreType.DMA,
    ],
)(_edge_body)


def _finalize_body(self_ref, a0_ref, a1_ref, d0_ref, d1_ref, out_ref):
    num = a0_ref[...] + a1_ref[...]
    den = jnp.sum(d0_ref[...] + d1_ref[...], axis=1, keepdims=True)
    recip = 8.0 / jnp.maximum(den, 1e-30)
    out_ref[...] = self_ref[...] + num * recip


def _finalize(selfh, a0, a1, d0, d1):
    nblk = N // _ROW_BLK
    return pl.pallas_call(
        _finalize_body,
        grid=(nblk,),
        in_specs=[
            pl.BlockSpec((_ROW_BLK, D), lambda i: (i, 0)),
            pl.BlockSpec((_ROW_BLK, D), lambda i: (i, 0)),
            pl.BlockSpec((_ROW_BLK, D), lambda i: (i, 0)),
            pl.BlockSpec((_ROW_BLK, 8), lambda i: (i, 0)),
            pl.BlockSpec((_ROW_BLK, 8), lambda i: (i, 0)),
        ],
        out_specs=pl.BlockSpec((_ROW_BLK, D), lambda i: (i, 0)),
        out_shape=jax.ShapeDtypeStruct((N, D), jnp.float32),
    )(selfh, a0, a1, d0, d1)


def kernel(ent_emb, edge_index, W_w, W_b, WS_w, WS_b, Q_w, Q_b, K_w, K_b):
    inv = jnp.float32(1.0 / jnp.sqrt(jnp.float32(D)))
    wqm = jnp.concatenate([Q_w.T * inv, W_w.T], axis=1)
    bqm = jnp.concatenate([Q_b * inv, W_b]).reshape(1, 2 * D)
    qmsg, k_all, selfh = _linears(ent_emb, wqm, bqm,
                                  K_w.T, K_b.reshape(1, D),
                                  WS_w.T, WS_b.reshape(1, D))
    src = edge_index[0]
    dst = edge_index[1]
    ei_flat = jnp.concatenate(
        [src.reshape(E // C, C), dst.reshape(E // C, C)], axis=1).reshape(-1)
    ei_flat = jnp.concatenate(
        [ei_flat, jnp.zeros(8 * C2, jnp.int32)])
    zeros = jnp.zeros((RPT, D), jnp.float32)
    qmsg_i = lax.bitcast_convert_type(qmsg.reshape(N, D, 2), jnp.int32)
    k_perm = jnp.take(k_all, _P, axis=1)
    acc, den = _edge_kernel(qmsg_i, k_perm, zeros, ei_flat)
    den_r = den.reshape(NC, NDEN * 16, 8)
    a0 = jnp.take(acc[0, :N], _PINV, axis=1)
    a1 = jnp.take(acc[1, :N], _PINV, axis=1)
    return _finalize(selfh, a0, a1, den_r[0, :N], den_r[1, :N])


# speedup vs baseline: 1.0500x; 1.0500x over previous
import functools

import jax
import jax.numpy as jnp
import numpy as np
from jax import lax
from jax.experimental import pallas as pl
from jax.experimental.pallas import tpu as pltpu
from jax.experimental.pallas import tpu_sc as plsc

N = 10000
E = 320000
D = 128

NC = 2
NS = 16
NW = NC * NS
EPW = E // NW
C = 40
C2 = 2 * C
NCHUNK = EPW // C

NPAD = 10112
RPT = NPAD // NS
NDEN = 640
DRPT = NDEN // NS

_ROW_BLK = 2000

_P = np.zeros(D, dtype=np.int32)
for _t in range(4):
    for _p in range(32):
        _P[32 * _t + _p] = 32 * _t + (2 * _p if _p < 16 else 2 * (_p - 16) + 1)
_PINV = np.argsort(_P).astype(np.int32)


def _linears_body(x_ref, wqm_ref, bqm_ref, wk_ref, bk_ref, ws_ref, bs_ref,
                  qmsg_ref, k_ref, self_ref):
    x = x_ref[...]
    qmsg_ref[...] = (jnp.dot(x, wqm_ref[...], preferred_element_type=jnp.float32)
                     + bqm_ref[...]).astype(jnp.bfloat16)
    k_ref[...] = jnp.dot(x, wk_ref[...],
                         preferred_element_type=jnp.float32) + bk_ref[...]
    self_ref[...] = jnp.dot(x, ws_ref[...],
                            preferred_element_type=jnp.float32) + bs_ref[...]


def _linears(x, wqm, bqm, wk, bk, ws, bs):
    nblk = N // _ROW_BLK
    return pl.pallas_call(
        _linears_body,
        grid=(nblk,),
        in_specs=[
            pl.BlockSpec((_ROW_BLK, D), lambda i: (i, 0)),
            pl.BlockSpec((D, 2 * D), lambda i: (0, 0)),
            pl.BlockSpec((1, 2 * D), lambda i: (0, 0)),
            pl.BlockSpec((D, D), lambda i: (0, 0)),
            pl.BlockSpec((1, D), lambda i: (0, 0)),
            pl.BlockSpec((D, D), lambda i: (0, 0)),
            pl.BlockSpec((1, D), lambda i: (0, 0)),
        ],
        out_specs=[
            pl.BlockSpec((_ROW_BLK, 2 * D), lambda i: (i, 0)),
            pl.BlockSpec((_ROW_BLK, D), lambda i: (i, 0)),
            pl.BlockSpec((_ROW_BLK, D), lambda i: (i, 0)),
        ],
        out_shape=[
            jax.ShapeDtypeStruct((N, 2 * D), jnp.bfloat16),
            jax.ShapeDtypeStruct((N, D), jnp.float32),
            jax.ShapeDtypeStruct((N, D), jnp.float32),
        ],
    )(x, wqm, bqm, wk, bk, ws, bs)


_GDN = lax.GatherDimensionNumbers(offset_dims=(), collapsed_slice_dims=(0,),
                                  start_index_map=(0,))


def _perm16(v, idx):
    return lax.gather(v, idx[:, None], _GDN, (1,), unique_indices=True,
                      mode=lax.GatherScatterMode.PROMISE_IN_BOUNDS)


def _halves(ref, e, t):
    xi = ref[e, pl.ds(16 * t, 16)]
    lo = lax.bitcast_convert_type(lax.shift_left(xi, 16), jnp.float32)
    hi = lax.bitcast_convert_type(jnp.bitwise_and(xi, jnp.int32(-65536)),
                                  jnp.float32)
    return lo, hi


def _edge_body(qmsg_hbm, k_hbm, zeros_hbm, ei_hbm,
               acc_out_hbm, den_out_hbm,
               ib0_, ib1_, oidx0, oidx1, qv0, qv1, kv0, kv1, oa0, oa1, arena,
               gq0, gq1, gk0, gk1, ss0, ss1):
    c = lax.axis_index("c")
    s = lax.axis_index("s")
    wid = s * NC + c
    row0 = wid * NCHUNK

    pltpu.sync_copy(zeros_hbm.at[pl.ds(0, RPT)],
                    arena.at[pl.ds(s * RPT, RPT)])
    pltpu.sync_copy(zeros_hbm.at[pl.ds(0, DRPT)],
                    arena.at[pl.ds(NPAD + s * DRPT, DRPT)])
    plsc.subcore_barrier()

    lane = lax.iota(jnp.int32, 16)
    perms = [lane ^ sh for sh in (1, 2, 4, 8)]
    zero16 = jnp.zeros((16,), jnp.float32)
    hi_m = lax.convert_element_type(lax.shift_right_logical(lane, 3),
                                    jnp.float32)
    mlo = 1.0 - hi_m
    mdiff = hi_m - mlo

    qvs = (qv0, qv1)
    kvs = (kv0, kv1)
    oidxs = (oidx0, oidx1)
    oas = (oa0, oa1)
    gqs = (gq0, gq1)
    gks = (gk0, gk1)
    sss = (ss0, ss1)

    ibs = (ib0_, ib1_)

    def _fetch_idx(i, b):
        pltpu.sync_copy(ei_hbm.at[pl.ds((row0 + i) * C2, C2)], ibs[b])

    def _gathers(i, b):
        return (
            pltpu.make_async_copy(qmsg_hbm.at[ibs[b].at[pl.ds(0, C)]],
                                  qvs[b], gqs[b]),
            pltpu.make_async_copy(k_hbm.at[ibs[b].at[pl.ds(C, C)]],
                                  kvs[b], gks[b]),
        )

    def _scatter_start(b):
        pltpu.async_copy(oas[b], arena.at[oidxs[b]], sss[b], add=True)

    def _scatter_wait(b):
        pltpu.make_async_copy(oas[b], arena.at[oidxs[b]], sss[b]).wait()

    def _stage(i, cur):
        nxt = 1 - cur

        @pl.when(i + 1 < NCHUNK)
        def _():
            _fetch_idx(i + 1, nxt)
            for cp in _gathers(i + 1, nxt):
                cp.start()

        @pl.when(i >= 2)
        def _():
            _scatter_wait(cur)

        for cp in _gathers(i, cur):
            cp.wait()
        ib = ibs[cur]
        qv = qvs[cur]
        kv = kvs[cur]
        oidx = oidxs[cur]
        o_all = oas[cur]
        for e0, u0 in ((0, 0), (16, 0), (24, 8)):
            dst16 = ib[pl.ds(C + e0, 16)]
            oidx[pl.ds(e0, 16)] = dst16
            oidx[pl.ds(C + e0, 16)] = lax.shift_right_logical(dst16, 4) + NPAD
            for u in range(u0, 16):
                e = e0 + u
                qlo, qhi = _halves(qv, e, 0)
                acc = (qlo * kv[e, pl.ds(0, 16)] +
                       qhi * kv[e, pl.ds(16, 16)])
                for t in range(1, 4):
                    qlo, qhi = _halves(qv, e, t)
                    acc = acc + (qlo * kv[e, pl.ds(32 * t, 16)] +
                                 qhi * kv[e, pl.ds(32 * t + 16, 16)])
                for p in perms:
                    acc = acc + _perm16(acc, p)
                ex = jnp.exp(acc)
                for t in range(4):
                    mlo_v, mhi_v = _halves(qv, e, 4 + t)
                    o_all[e, pl.ds(32 * t, 16)] = ex * mlo_v
                    o_all[e, pl.ds(32 * t + 16, 16)] = ex * mhi_v
                dst_s = dst16[u]
                halff = lax.convert_element_type(
                    jnp.bitwise_and(dst_s, 1), jnp.float32)
                vec = ex * (mlo + halff * mdiff)
                alig = pl.multiple_of(
                    jnp.bitwise_and(lax.shift_right_logical(dst_s, 1), 7)
                    * 16, 16)
                for j in range(8):
                    o_all[C + e, pl.ds(16 * j, 16)] = zero16
                o_all[C + e, pl.ds(alig, 16)] = vec
        _scatter_start(cur)

    _fetch_idx(0, 0)
    for cp in _gathers(0, 0):
        cp.start()

    def pair(t, carry):
        _stage(2 * t, 0)
        _stage(2 * t + 1, 1)
        return carry

    lax.fori_loop(0, NCHUNK // 2, pair, 0)
    _scatter_wait(0)
    _scatter_wait(1)

    plsc.subcore_barrier()
    pltpu.sync_copy(arena.at[pl.ds(s * RPT, RPT)],
                    acc_out_hbm.at[c, pl.ds(s * RPT, RPT)])
    pltpu.sync_copy(arena.at[pl.ds(NPAD + s * DRPT, DRPT)],
                    den_out_hbm.at[c, pl.ds(s * DRPT, DRPT)])


_edge_kernel = functools.partial(
    pl.kernel,
    out_type=[
        jax.ShapeDtypeStruct((NC, NPAD, D), jnp.float32),
        jax.ShapeDtypeStruct((NC, NDEN, D), jnp.float32),
    ],
    mesh=plsc.VectorSubcoreMesh(core_axis_name="c", subcore_axis_name="s"),
    scratch_types=[
        pltpu.VMEM((C2,), jnp.int32),
        pltpu.VMEM((C2,), jnp.int32),
        pltpu.VMEM((C2,), jnp.int32),
        pltpu.VMEM((C2,), jnp.int32),
        pltpu.VMEM((C, D), jnp.int32),
        pltpu.VMEM((C, D), jnp.int32),
        pltpu.VMEM((C, D), jnp.float32),
        pltpu.VMEM((C, D), jnp.float32),
        pltpu.VMEM((C2, D), jnp.float32),
        pltpu.VMEM((C2, D), jnp.float32),
        pltpu.VMEM_SHARED((NPAD + NDEN, D), jnp.float32),
        pltpu.SemaphoreType.DMA,
        pltpu.SemaphoreType.DMA,
        pltpu.SemaphoreType.DMA,
        pltpu.SemaphoreType.DMA,
        pltpu.SemaphoreType.DMA,
        pltpu.SemaphoreType.DMA,
    ],
)(_edge_body)


def _finalize_body(self_ref, a0_ref, a1_ref, d0_ref, d1_ref, out_ref):
    num = a0_ref[...] + a1_ref[...]
    den = jnp.sum(d0_ref[...] + d1_ref[...], axis=1, keepdims=True)
    recip = 8.0 / jnp.maximum(den, 1e-30)
    out_ref[...] = self_ref[...] + num * recip


def _finalize(selfh, a0, a1, d0, d1):
    nblk = N // _ROW_BLK
    return pl.pallas_call(
        _finalize_body,
        grid=(nblk,),
        in_specs=[
            pl.BlockSpec((_ROW_BLK, D), lambda i: (i, 0)),
            pl.BlockSpec((_ROW_BLK, D), lambda i: (i, 0)),
            pl.BlockSpec((_ROW_BLK, D), lambda i: (i, 0)),
            pl.BlockSpec((_ROW_BLK, 8), lambda i: (i, 0)),
            pl.BlockSpec((_ROW_BLK, 8), lambda i: (i, 0)),
        ],
        out_specs=pl.BlockSpec((_ROW_BLK, D), lambda i: (i, 0)),
        out_shape=jax.ShapeDtypeStruct((N, D), jnp.float32),
    )(selfh, a0, a1, d0, d1)


def kernel(ent_emb, edge_index, W_w, W_b, WS_w, WS_b, Q_w, Q_b, K_w, K_b):
    inv = jnp.float32(1.0 / jnp.sqrt(jnp.float32(D)))
    wqm = jnp.concatenate([Q_w.T * inv, W_w.T], axis=1)
    bqm = jnp.concatenate([Q_b * inv, W_b]).reshape(1, 2 * D)
    qmsg, k_all, selfh = _linears(ent_emb, wqm, bqm,
                                  K_w.T, K_b.reshape(1, D),
                                  WS_w.T, WS_b.reshape(1, D))
    src = edge_index[0]
    dst = edge_index[1]
    ei_flat = jnp.concatenate(
        [src.reshape(E // C, C), dst.reshape(E // C, C)], axis=1).reshape(-1)
    zeros = jnp.zeros((RPT, D), jnp.float32)
    qmsg_i = lax.bitcast_convert_type(qmsg.reshape(N, D, 2), jnp.int32)
    k_perm = jnp.take(k_all, _P, axis=1)
    acc, den = _edge_kernel(qmsg_i, k_perm, zeros, ei_flat)
    den_r = den.reshape(NC, NDEN * 16, 8)
    a0 = jnp.take(acc[0, :N], _PINV, axis=1)
    a1 = jnp.take(acc[1, :N], _PINV, axis=1)
    return _finalize(selfh, a0, a1, den_r[0, :N], den_r[1, :N])
